# baseline (speedup 1.0000x reference)
import functools

import jax
import jax.numpy as jnp
from jax import lax
from jax.experimental import pallas as pl
from jax.experimental.pallas import tpu as pltpu
from jax.experimental.pallas import tpu_sc as plsc

PROTO_NUM = 4
N_CLASSES = 1000
FEAT_DIM = 512
BATCH = 16384
ROWS = PROTO_NUM * N_CLASSES

L = 16
NC = 2
NS = 16
NW = NC * NS
SC_BATCH = 9216
TC_BLK = 1024
TC_GRID = (BATCH - SC_BATCH) // TC_BLK
PER_W = SC_BATCH // NW
CHUNK = 32
NCHUNK = PER_W // CHUNK
NBUF = 3
GROUP = BATCH // PROTO_NUM
HALF = FEAT_DIM // 2
WPR = HALF // L


def _prep_body(table_ref, packed_ref, tb_ref):
    t = table_ref[...]
    n = jnp.sqrt(jnp.sum(t * t, axis=1, keepdims=True))
    tn = t / jnp.maximum(n, jnp.float32(1e-12))
    tb16 = tn.astype(jnp.bfloat16)
    tb_ref[...] = tb16
    tb = tb16.astype(jnp.float32)
    lo = lax.bitcast_convert_type(tb[:, :HALF], jnp.int32)
    hi = lax.bitcast_convert_type(tb[:, HALF:], jnp.int32)
    packed = lax.bitwise_or(
        lax.shift_right_logical(lo, 16),
        lax.bitwise_and(hi, jnp.int32(-65536)))
    packed_ref[...] = lax.bitcast_convert_type(packed, jnp.float32)


def _tc_half_body(f_ref, y_ref, tb_ref, out_ref):
    i = pl.program_id(0)
    f = f_ref[...]
    yv = y_ref[...]
    cols = lax.broadcasted_iota(jnp.int32, (TC_BLK, N_CLASSES), 1)
    oh = (cols == yv[:, None]).astype(jnp.bfloat16)
    k = jnp.dot(oh, tb_ref[...], preferred_element_type=jnp.float32)
    ff = jnp.sum(f * f, axis=1)
    fk = jnp.sum(f * k, axis=1)
    cos = fk * lax.rsqrt(jnp.maximum(ff, jnp.float32(1e-35)))
    d = jnp.sqrt(jnp.maximum(jnp.float32(2.0) - jnp.float32(2.0) * cos,
                             jnp.float32(0.0)))
    psum = jnp.sum(d) / jnp.float32(8 * 128)

    @pl.when(i == 0)
    def _():
        out_ref[...] = jnp.zeros((8, 128), jnp.float32)

    out_ref[...] = out_ref[...] + jnp.broadcast_to(psum, (8, 128))


def _rsqrt(x):
    i = lax.bitcast_convert_type(x, jnp.int32)
    i = jnp.int32(0x5F3759DF) - lax.shift_right_arithmetic(i, 1)
    y = lax.bitcast_convert_type(i, jnp.float32)
    for _ in range(2):
        y = y * (jnp.float32(1.5) - jnp.float32(0.5) * x * y * y)
    return y


def _sqrt(x):
    return x * _rsqrt(jnp.maximum(x, jnp.float32(1e-35)))


def _sc_body(feat_hbm, y_hbm, table_hbm, out_hbm,
             idx_v, loss_v, f0, f1, f2, k0, k1, k2,
             sf0, sf1, sf2, sk0, sk1, sk2):
    cid = lax.axis_index("c")
    sid = lax.axis_index("s")
    wid = sid * NC + cid
    base = wid * PER_W

    pltpu.sync_copy(y_hbm.at[pl.ds(base, PER_W)], idx_v)
    lanei = lax.iota(jnp.int32, L)
    for j in range(PER_W // L):
        sl = pl.ds(j * L, L)
        sids = lanei + (base + j * L)
        grp = lax.shift_right_logical(sids, 12)
        idx_v[sl] = idx_v[sl] + grp * jnp.int32(N_CLASSES)

    fbufs = (f0, f1, f2)
    kbufs = (k0, k1, k2)
    fsems = (sf0, sf1, sf2)
    ksems = (sk0, sk1, sk2)

    def issue(c):
        b = c % NBUF
        fcp = pltpu.async_copy(
            feat_hbm.at[pl.ds(base + c * CHUNK, CHUNK)], fbufs[b], fsems[b])
        kcp = pltpu.async_copy(
            table_hbm.at[idx_v.at[pl.ds(c * CHUNK, CHUNK)]], kbufs[b], ksems[b])
        return fcp, kcp

    pend = [issue(0), issue(1)]

    lane = lax.iota(jnp.int32, L)
    perms = [lax.bitwise_xor(lane, jnp.int32(sh)) for sh in (8, 4, 2, 1)]
    dnums = lax.GatherDimensionNumbers(
        offset_dims=(), collapsed_slice_dims=(0,), start_index_map=(0,))

    def shuffle(x, p):
        return lax.gather(
            x, p[:, None], dnums, (1,),
            mode=lax.GatherScatterMode.PROMISE_IN_BOUNDS)

    def lanesum(x):
        for p in perms:
            x = x + shuffle(x, p)
        return x

    himask = jnp.int32(-65536)

    def compute_chunk(c, acc):
        fb = fbufs[c % NBUF]
        kb = kbufs[c % NBUF]

        def dist(s):
            ff = jnp.zeros((L,), jnp.float32)
            fk = jnp.zeros((L,), jnp.float32)
            for j in range(WPR):
                kw = lax.bitcast_convert_type(
                    kb[s, pl.ds(j * L, L)], jnp.int32)
                klo = lax.bitcast_convert_type(
                    lax.shift_left(kw, jnp.int32(16)), jnp.float32)
                khi = lax.bitcast_convert_type(
                    lax.bitwise_and(kw, himask), jnp.float32)
                flo = fb[s, pl.ds(j * L, L)]
                fhi = fb[s, pl.ds(HALF + j * L, L)]
                ff = ff + flo * flo + fhi * fhi
                fk = fk + flo * klo + fhi * khi
            ffs = lanesum(ff)
            fks = lanesum(fk)
            inv = _rsqrt(jnp.maximum(ffs, jnp.float32(1e-35)))
            cos = fks * inv
            d2 = jnp.maximum(jnp.float32(2.0) - jnp.float32(2.0) * cos,
                             jnp.float32(0.0))
            return _sqrt(d2)

        def sample(s, a):
            return a + dist(s)

        return lax.fori_loop(0, CHUNK, sample, acc)

    acc = jnp.zeros((L,), jnp.float32)
    for c in range(NCHUNK):
        fcp, kcp = pend[0]
        fcp.wait()
        kcp.wait()
        pend = pend[1:]
        if c + 2 < NCHUNK:
            pend.append(issue(c + 2))
        acc = compute_chunk(c, acc)

    loss_v[...] = acc
    pltpu.sync_copy(loss_v, out_hbm.at[wid])


@jax.jit
def kernel(feature, y, y_pred, prototype):
    del y_pred
    table = jnp.reshape(prototype, (ROWS, FEAT_DIM))

    prep_rows = ROWS // 2
    table_pk, table_bf = pl.pallas_call(
        _prep_body,
        grid=(2,),
        in_specs=[pl.BlockSpec((prep_rows, FEAT_DIM), lambda i: (i, 0))],
        out_specs=[
            pl.BlockSpec((prep_rows, HALF), lambda i: (i, 0)),
            pl.BlockSpec((prep_rows, FEAT_DIM), lambda i: (i, 0)),
        ],
        out_shape=[
            jax.ShapeDtypeStruct((ROWS, HALF), jnp.float32),
            jax.ShapeDtypeStruct((ROWS, FEAT_DIM), jnp.bfloat16),
        ],
    )(table)

    tc_part = pl.pallas_call(
        _tc_half_body,
        grid=(TC_GRID,),
        in_specs=[
            pl.BlockSpec((TC_BLK, FEAT_DIM),
                         lambda i: (SC_BATCH // TC_BLK + i, 0)),
            pl.BlockSpec((TC_BLK,), lambda i: (SC_BATCH // TC_BLK + i,)),
            pl.BlockSpec((N_CLASSES, FEAT_DIM),
                         lambda i: ((SC_BATCH // TC_BLK + i) // (GROUP // TC_BLK), 0)),
        ],
        out_specs=pl.BlockSpec((8, 128), lambda i: (0, 0)),
        out_shape=jax.ShapeDtypeStruct((8, 128), jnp.float32),
    )(feature, y, table_bf)

    mesh = plsc.VectorSubcoreMesh(core_axis_name="c", subcore_axis_name="s")
    partial = pl.kernel(
        _sc_body,
        out_type=jax.ShapeDtypeStruct((NW, L), jnp.float32),
        mesh=mesh,
        compiler_params=pltpu.CompilerParams(needs_layout_passes=False),
        scratch_types=[
            pltpu.VMEM((PER_W,), jnp.int32),
            pltpu.VMEM((L,), jnp.float32),
            pltpu.VMEM((CHUNK, FEAT_DIM), jnp.float32),
            pltpu.VMEM((CHUNK, FEAT_DIM), jnp.float32),
            pltpu.VMEM((CHUNK, FEAT_DIM), jnp.float32),
            pltpu.VMEM((CHUNK, HALF), jnp.float32),
            pltpu.VMEM((CHUNK, HALF), jnp.float32),
            pltpu.VMEM((CHUNK, HALF), jnp.float32),
            pltpu.SemaphoreType.DMA,
            pltpu.SemaphoreType.DMA,
            pltpu.SemaphoreType.DMA,
            pltpu.SemaphoreType.DMA,
            pltpu.SemaphoreType.DMA,
            pltpu.SemaphoreType.DMA,
        ],
    )(feature, y, table_pk)
    parts = jnp.concatenate([
        jnp.reshape(partial, (-1,)) * jnp.float32(1.0 / L),
        jnp.reshape(tc_part, (-1,)),
    ])
    return jnp.sum(parts) / jnp.float32(BATCH)


# speedup vs baseline: 1.0210x; 1.0210x over previous
import functools

import jax
import jax.numpy as jnp
from jax import lax
from jax.experimental import pallas as pl
from jax.experimental.pallas import tpu as pltpu
from jax.experimental.pallas import tpu_sc as plsc

PROTO_NUM = 4
N_CLASSES = 1000
FEAT_DIM = 512
BATCH = 16384
ROWS = PROTO_NUM * N_CLASSES

L = 16
NC = 2
NS = 16
NW = NC * NS
SC_BATCH = 8192
TC_BLK = 1024
TC_GRID = (BATCH - SC_BATCH) // TC_BLK
PER_W = SC_BATCH // NW
CHUNK = 32
NCHUNK = PER_W // CHUNK
NBUF = 3
GROUP = BATCH // PROTO_NUM
HALF = FEAT_DIM // 2
WPR = HALF // L


def _prep_body(table_ref, packed_ref, tb_ref):
    t = table_ref[...]
    n = jnp.sqrt(jnp.sum(t * t, axis=1, keepdims=True))
    tn = t / jnp.maximum(n, jnp.float32(1e-12))
    tb16 = tn.astype(jnp.bfloat16)
    tb_ref[...] = tb16
    tb = tb16.astype(jnp.float32)
    lo = lax.bitcast_convert_type(tb[:, :HALF], jnp.int32)
    hi = lax.bitcast_convert_type(tb[:, HALF:], jnp.int32)
    packed = lax.bitwise_or(
        lax.shift_right_logical(lo, 16),
        lax.bitwise_and(hi, jnp.int32(-65536)))
    packed_ref[...] = lax.bitcast_convert_type(packed, jnp.float32)


def _tc_half_body(f_ref, y_ref, tb_ref, out_ref):
    i = pl.program_id(0)
    f = f_ref[...]
    yv = y_ref[...]
    cols = lax.broadcasted_iota(jnp.int32, (TC_BLK, N_CLASSES), 1)
    oh = (cols == yv[:, None]).astype(jnp.bfloat16)
    k = jnp.dot(oh, tb_ref[...], preferred_element_type=jnp.float32)
    ff = jnp.sum(f * f, axis=1)
    fk = jnp.sum(f * k, axis=1)
    cos = fk * lax.rsqrt(jnp.maximum(ff, jnp.float32(1e-35)))
    d = jnp.sqrt(jnp.maximum(jnp.float32(2.0) - jnp.float32(2.0) * cos,
                             jnp.float32(0.0)))
    psum = jnp.sum(d) / jnp.float32(8 * 128)

    @pl.when(i == 0)
    def _():
        out_ref[...] = jnp.zeros((8, 128), jnp.float32)

    out_ref[...] = out_ref[...] + jnp.broadcast_to(psum, (8, 128))


def _rsqrt(x):
    i = lax.bitcast_convert_type(x, jnp.int32)
    i = jnp.int32(0x5F3759DF) - lax.shift_right_arithmetic(i, 1)
    y = lax.bitcast_convert_type(i, jnp.float32)
    for _ in range(2):
        y = y * (jnp.float32(1.5) - jnp.float32(0.5) * x * y * y)
    return y


def _sqrt(x):
    return x * _rsqrt(jnp.maximum(x, jnp.float32(1e-35)))


def _sc_body(feat_hbm, y_hbm, table_hbm, out_hbm,
             idx_v, loss_v, f0, f1, f2, k0, k1, k2,
             sf0, sf1, sf2, sk0, sk1, sk2):
    cid = lax.axis_index("c")
    sid = lax.axis_index("s")
    wid = sid * NC + cid
    base = wid * PER_W

    pltpu.sync_copy(y_hbm.at[pl.ds(base, PER_W)], idx_v)
    lanei = lax.iota(jnp.int32, L)
    for j in range(PER_W // L):
        sl = pl.ds(j * L, L)
        sids = lanei + (base + j * L)
        grp = lax.shift_right_logical(sids, 12)
        idx_v[sl] = idx_v[sl] + grp * jnp.int32(N_CLASSES)

    fbufs = (f0, f1, f2)
    kbufs = (k0, k1, k2)
    fsems = (sf0, sf1, sf2)
    ksems = (sk0, sk1, sk2)

    def issue(c):
        b = c % NBUF
        fcp = pltpu.async_copy(
            feat_hbm.at[pl.ds(base + c * CHUNK, CHUNK)], fbufs[b], fsems[b])
        kcp = pltpu.async_copy(
            table_hbm.at[idx_v.at[pl.ds(c * CHUNK, CHUNK)]], kbufs[b], ksems[b])
        return fcp, kcp

    pend = [issue(0), issue(1)]

    lane = lax.iota(jnp.int32, L)
    perms = [lax.bitwise_xor(lane, jnp.int32(sh)) for sh in (8, 4, 2, 1)]
    dnums = lax.GatherDimensionNumbers(
        offset_dims=(), collapsed_slice_dims=(0,), start_index_map=(0,))

    def shuffle(x, p):
        return lax.gather(
            x, p[:, None], dnums, (1,),
            mode=lax.GatherScatterMode.PROMISE_IN_BOUNDS)

    def lanesum(x):
        for p in perms:
            x = x + shuffle(x, p)
        return x

    himask = jnp.int32(-65536)

    def compute_chunk(c, acc):
        fb = fbufs[c % NBUF]
        kb = kbufs[c % NBUF]

        def dist(s):
            ff = jnp.zeros((L,), jnp.float32)
            fk = jnp.zeros((L,), jnp.float32)
            for j in range(WPR):
                kw = lax.bitcast_convert_type(
                    kb[s, pl.ds(j * L, L)], jnp.int32)
                klo = lax.bitcast_convert_type(
                    lax.shift_left(kw, jnp.int32(16)), jnp.float32)
                khi = lax.bitcast_convert_type(
                    lax.bitwise_and(kw, himask), jnp.float32)
                flo = fb[s, pl.ds(j * L, L)]
                fhi = fb[s, pl.ds(HALF + j * L, L)]
                ff = ff + flo * flo + fhi * fhi
                fk = fk + flo * klo + fhi * khi
            ffs = lanesum(ff)
            fks = lanesum(fk)
            inv = _rsqrt(jnp.maximum(ffs, jnp.float32(1e-35)))
            cos = fks * inv
            d2 = jnp.maximum(jnp.float32(2.0) - jnp.float32(2.0) * cos,
                             jnp.float32(0.0))
            return _sqrt(d2)

        def sample(s, a):
            return a + dist(s)

        return lax.fori_loop(0, CHUNK, sample, acc)

    acc = jnp.zeros((L,), jnp.float32)
    for c in range(NCHUNK):
        fcp, kcp = pend[0]
        fcp.wait()
        kcp.wait()
        pend = pend[1:]
        if c + 2 < NCHUNK:
            pend.append(issue(c + 2))
        acc = compute_chunk(c, acc)

    loss_v[...] = acc
    pltpu.sync_copy(loss_v, out_hbm.at[wid])


@jax.jit
def kernel(feature, y, y_pred, prototype):
    del y_pred
    table = jnp.reshape(prototype, (ROWS, FEAT_DIM))

    prep_rows = ROWS // 2
    table_pk, table_bf = pl.pallas_call(
        _prep_body,
        grid=(2,),
        in_specs=[pl.BlockSpec((prep_rows, FEAT_DIM), lambda i: (i, 0))],
        out_specs=[
            pl.BlockSpec((prep_rows, HALF), lambda i: (i, 0)),
            pl.BlockSpec((prep_rows, FEAT_DIM), lambda i: (i, 0)),
        ],
        out_shape=[
            jax.ShapeDtypeStruct((ROWS, HALF), jnp.float32),
            jax.ShapeDtypeStruct((ROWS, FEAT_DIM), jnp.bfloat16),
        ],
    )(table)

    tc_part = pl.pallas_call(
        _tc_half_body,
        grid=(TC_GRID,),
        in_specs=[
            pl.BlockSpec((TC_BLK, FEAT_DIM),
                         lambda i: (SC_BATCH // TC_BLK + i, 0)),
            pl.BlockSpec((TC_BLK,), lambda i: (SC_BATCH // TC_BLK + i,)),
            pl.BlockSpec((N_CLASSES, FEAT_DIM),
                         lambda i: ((SC_BATCH // TC_BLK + i) // (GROUP // TC_BLK), 0)),
        ],
        out_specs=pl.BlockSpec((8, 128), lambda i: (0, 0)),
        out_shape=jax.ShapeDtypeStruct((8, 128), jnp.float32),
    )(feature, y, table_bf)

    mesh = plsc.VectorSubcoreMesh(core_axis_name="c", subcore_axis_name="s")
    partial = pl.kernel(
        _sc_body,
        out_type=jax.ShapeDtypeStruct((NW, L), jnp.float32),
        mesh=mesh,
        compiler_params=pltpu.CompilerParams(needs_layout_passes=False),
        scratch_types=[
            pltpu.VMEM((PER_W,), jnp.int32),
            pltpu.VMEM((L,), jnp.float32),
            pltpu.VMEM((CHUNK, FEAT_DIM), jnp.float32),
            pltpu.VMEM((CHUNK, FEAT_DIM), jnp.float32),
            pltpu.VMEM((CHUNK, FEAT_DIM), jnp.float32),
            pltpu.VMEM((CHUNK, HALF), jnp.float32),
            pltpu.VMEM((CHUNK, HALF), jnp.float32),
            pltpu.VMEM((CHUNK, HALF), jnp.float32),
            pltpu.SemaphoreType.DMA,
            pltpu.SemaphoreType.DMA,
            pltpu.SemaphoreType.DMA,
            pltpu.SemaphoreType.DMA,
            pltpu.SemaphoreType.DMA,
            pltpu.SemaphoreType.DMA,
        ],
    )(feature, y, table_pk)
    parts = jnp.concatenate([
        jnp.reshape(partial, (-1,)) * jnp.float32(1.0 / L),
        jnp.reshape(tc_part, (-1,)),
    ])
    return jnp.sum(parts) / jnp.float32(BATCH)


# speedup vs baseline: 1.0364x; 1.0151x over previous
import functools

import jax
import jax.numpy as jnp
from jax import lax
from jax.experimental import pallas as pl
from jax.experimental.pallas import tpu as pltpu
from jax.experimental.pallas import tpu_sc as plsc

PROTO_NUM = 4
N_CLASSES = 1000
FEAT_DIM = 512
BATCH = 16384
ROWS = PROTO_NUM * N_CLASSES

L = 16
NC = 2
NS = 16
NW = NC * NS
SC_BATCH = 8192
TC_BLK = 2048
TC_GRID = (BATCH - SC_BATCH) // TC_BLK
PER_W = SC_BATCH // NW
CHUNK = 32
NCHUNK = PER_W // CHUNK
NBUF = 3
GROUP = BATCH // PROTO_NUM
HALF = FEAT_DIM // 2
WPR = HALF // L


def _prep_body(table_ref, packed_ref, tb_ref):
    t = table_ref[...]
    n = jnp.sqrt(jnp.sum(t * t, axis=1, keepdims=True))
    tn = t / jnp.maximum(n, jnp.float32(1e-12))
    tb16 = tn.astype(jnp.bfloat16)
    tb_ref[...] = tb16
    tb = tb16.astype(jnp.float32)
    lo = lax.bitcast_convert_type(tb[:, :HALF], jnp.int32)
    hi = lax.bitcast_convert_type(tb[:, HALF:], jnp.int32)
    packed = lax.bitwise_or(
        lax.shift_right_logical(lo, 16),
        lax.bitwise_and(hi, jnp.int32(-65536)))
    packed_ref[...] = lax.bitcast_convert_type(packed, jnp.float32)


def _tc_half_body(f_ref, y_ref, tb_ref, out_ref):
    i = pl.program_id(0)
    f = f_ref[...]
    yv = y_ref[...]
    cols = lax.broadcasted_iota(jnp.int32, (TC_BLK, N_CLASSES), 1)
    oh = (cols == yv[:, None]).astype(jnp.bfloat16)
    k = jnp.dot(oh, tb_ref[...], preferred_element_type=jnp.float32)
    ff = jnp.sum(f * f, axis=1)
    fk = jnp.sum(f * k, axis=1)
    cos = fk * lax.rsqrt(jnp.maximum(ff, jnp.float32(1e-35)))
    d = jnp.sqrt(jnp.maximum(jnp.float32(2.0) - jnp.float32(2.0) * cos,
                             jnp.float32(0.0)))
    psum = jnp.sum(d) / jnp.float32(8 * 128)

    @pl.when(i == 0)
    def _():
        out_ref[...] = jnp.zeros((8, 128), jnp.float32)

    out_ref[...] = out_ref[...] + jnp.broadcast_to(psum, (8, 128))


def _rsqrt(x):
    i = lax.bitcast_convert_type(x, jnp.int32)
    i = jnp.int32(0x5F3759DF) - lax.shift_right_arithmetic(i, 1)
    y = lax.bitcast_convert_type(i, jnp.float32)
    for _ in range(2):
        y = y * (jnp.float32(1.5) - jnp.float32(0.5) * x * y * y)
    return y


def _sqrt(x):
    return x * _rsqrt(jnp.maximum(x, jnp.float32(1e-35)))


def _sc_body(feat_hbm, y_hbm, table_hbm, out_hbm,
             idx_v, loss_v, f0, f1, f2, k0, k1, k2,
             sf0, sf1, sf2, sk0, sk1, sk2):
    cid = lax.axis_index("c")
    sid = lax.axis_index("s")
    wid = sid * NC + cid
    base = wid * PER_W

    pltpu.sync_copy(y_hbm.at[pl.ds(base, PER_W)], idx_v)
    lanei = lax.iota(jnp.int32, L)
    for j in range(PER_W // L):
        sl = pl.ds(j * L, L)
        sids = lanei + (base + j * L)
        grp = lax.shift_right_logical(sids, 12)
        idx_v[sl] = idx_v[sl] + grp * jnp.int32(N_CLASSES)

    fbufs = (f0, f1, f2)
    kbufs = (k0, k1, k2)
    fsems = (sf0, sf1, sf2)
    ksems = (sk0, sk1, sk2)

    def issue(c):
        b = c % NBUF
        fcp = pltpu.async_copy(
            feat_hbm.at[pl.ds(base + c * CHUNK, CHUNK)], fbufs[b], fsems[b])
        kcp = pltpu.async_copy(
            table_hbm.at[idx_v.at[pl.ds(c * CHUNK, CHUNK)]], kbufs[b], ksems[b])
        return fcp, kcp

    pend = [issue(0), issue(1)]

    lane = lax.iota(jnp.int32, L)
    perms = [lax.bitwise_xor(lane, jnp.int32(sh)) for sh in (8, 4, 2, 1)]
    dnums = lax.GatherDimensionNumbers(
        offset_dims=(), collapsed_slice_dims=(0,), start_index_map=(0,))

    def shuffle(x, p):
        return lax.gather(
            x, p[:, None], dnums, (1,),
            mode=lax.GatherScatterMode.PROMISE_IN_BOUNDS)

    def lanesum(x):
        for p in perms:
            x = x + shuffle(x, p)
        return x

    himask = jnp.int32(-65536)

    def compute_chunk(c, acc):
        fb = fbufs[c % NBUF]
        kb = kbufs[c % NBUF]

        def dist(s):
            ff = jnp.zeros((L,), jnp.float32)
            fk = jnp.zeros((L,), jnp.float32)
            for j in range(WPR):
                kw = lax.bitcast_convert_type(
                    kb[s, pl.ds(j * L, L)], jnp.int32)
                klo = lax.bitcast_convert_type(
                    lax.shift_left(kw, jnp.int32(16)), jnp.float32)
                khi = lax.bitcast_convert_type(
                    lax.bitwise_and(kw, himask), jnp.float32)
                flo = fb[s, pl.ds(j * L, L)]
                fhi = fb[s, pl.ds(HALF + j * L, L)]
                ff = ff + flo * flo + fhi * fhi
                fk = fk + flo * klo + fhi * khi
            ffs = lanesum(ff)
            fks = lanesum(fk)
            inv = _rsqrt(jnp.maximum(ffs, jnp.float32(1e-35)))
            cos = fks * inv
            d2 = jnp.maximum(jnp.float32(2.0) - jnp.float32(2.0) * cos,
                             jnp.float32(0.0))
            return _sqrt(d2)

        def sample(s, a):
            return a + dist(s)

        return lax.fori_loop(0, CHUNK, sample, acc)

    acc = jnp.zeros((L,), jnp.float32)
    for c in range(NCHUNK):
        fcp, kcp = pend[0]
        fcp.wait()
        kcp.wait()
        pend = pend[1:]
        if c + 2 < NCHUNK:
            pend.append(issue(c + 2))
        acc = compute_chunk(c, acc)

    loss_v[...] = acc
    pltpu.sync_copy(loss_v, out_hbm.at[wid])


@jax.jit
def kernel(feature, y, y_pred, prototype):
    del y_pred
    table = jnp.reshape(prototype, (ROWS, FEAT_DIM))

    prep_rows = ROWS // 2
    table_pk, table_bf = pl.pallas_call(
        _prep_body,
        grid=(2,),
        in_specs=[pl.BlockSpec((prep_rows, FEAT_DIM), lambda i: (i, 0))],
        out_specs=[
            pl.BlockSpec((prep_rows, HALF), lambda i: (i, 0)),
            pl.BlockSpec((prep_rows, FEAT_DIM), lambda i: (i, 0)),
        ],
        out_shape=[
            jax.ShapeDtypeStruct((ROWS, HALF), jnp.float32),
            jax.ShapeDtypeStruct((ROWS, FEAT_DIM), jnp.bfloat16),
        ],
    )(table)

    tc_part = pl.pallas_call(
        _tc_half_body,
        grid=(TC_GRID,),
        in_specs=[
            pl.BlockSpec((TC_BLK, FEAT_DIM),
                         lambda i: (SC_BATCH // TC_BLK + i, 0)),
            pl.BlockSpec((TC_BLK,), lambda i: (SC_BATCH // TC_BLK + i,)),
            pl.BlockSpec((N_CLASSES, FEAT_DIM),
                         lambda i: ((SC_BATCH // TC_BLK + i) // (GROUP // TC_BLK), 0)),
        ],
        out_specs=pl.BlockSpec((8, 128), lambda i: (0, 0)),
        out_shape=jax.ShapeDtypeStruct((8, 128), jnp.float32),
    )(feature, y, table_bf)

    mesh = plsc.VectorSubcoreMesh(core_axis_name="c", subcore_axis_name="s")
    partial = pl.kernel(
        _sc_body,
        out_type=jax.ShapeDtypeStruct((NW, L), jnp.float32),
        mesh=mesh,
        compiler_params=pltpu.CompilerParams(needs_layout_passes=False),
        scratch_types=[
            pltpu.VMEM((PER_W,), jnp.int32),
            pltpu.VMEM((L,), jnp.float32),
            pltpu.VMEM((CHUNK, FEAT_DIM), jnp.float32),
            pltpu.VMEM((CHUNK, FEAT_DIM), jnp.float32),
            pltpu.VMEM((CHUNK, FEAT_DIM), jnp.float32),
            pltpu.VMEM((CHUNK, HALF), jnp.float32),
            pltpu.VMEM((CHUNK, HALF), jnp.float32),
            pltpu.VMEM((CHUNK, HALF), jnp.float32),
            pltpu.SemaphoreType.DMA,
            pltpu.SemaphoreType.DMA,
            pltpu.SemaphoreType.DMA,
            pltpu.SemaphoreType.DMA,
            pltpu.SemaphoreType.DMA,
            pltpu.SemaphoreType.DMA,
        ],
    )(feature, y, table_pk)
    parts = jnp.concatenate([
        jnp.reshape(partial, (-1,)) * jnp.float32(1.0 / L),
        jnp.reshape(tc_part, (-1,)),
    ])
    return jnp.sum(parts) / jnp.float32(BATCH)
